# baseline (speedup 1.0000x reference)
import math

import jax
import jax.numpy as jnp
from jax import lax
from jax.experimental import pallas as pl
from jax.experimental.pallas import tpu as pltpu
import ml_dtypes
import numpy as np

_KSIZE = 5
_SIGMA = 1.4
_LOW_T = 0.1
_HIGH_T = 0.3
_MAX_HYST_ITERS = 256
_T1 = math.tan(math.radians(22.5))
_T2 = math.tan(math.radians(67.5))


def _bf16_round(v):
    return float(np.float32(np.float32(v).astype(ml_dtypes.bfloat16)))


def _gauss2d_bf16():
    c = (_KSIZE - 1) / 2.0
    g = np.exp(-((np.arange(_KSIZE) - c) ** 2) / (2.0 * _SIGMA * _SIGMA))
    g = g / g.sum()
    k = np.outer(g, g).astype(np.float32)
    return [[_bf16_round(k[i, j]) for j in range(_KSIZE)] for i in range(_KSIZE)]


_GRAY_W = [_bf16_round(v) for v in (0.299, 0.587, 0.114)]


def _bf(t):
    return t.astype(jnp.bfloat16).astype(jnp.float32)


def _reflect_pad(a, ph, pw):
    if ph:
        top = [a[ph - i:ph + 1 - i] for i in range(ph)]
        bot = [a[a.shape[0] - 2 - i:a.shape[0] - 1 - i] for i in range(ph)]
        a = jnp.concatenate(top + [a] + bot, axis=0)
    if pw:
        left = [a[:, pw - i:pw + 1 - i] for i in range(pw)]
        right = [a[:, a.shape[1] - 2 - i:a.shape[1] - 1 - i] for i in range(pw)]
        a = jnp.concatenate(left + [a] + right, axis=1)
    return a


def _zero_pad1(a):
    H, W = a.shape
    z_row = jnp.zeros((1, W), a.dtype)
    a = jnp.concatenate([z_row, a, z_row], axis=0)
    z_col = jnp.zeros((H + 2, 1), a.dtype)
    return jnp.concatenate([z_col, a, z_col], axis=1)


def _nms_stage(xb, H, W):
    gray = (_bf(xb[0]) * _GRAY_W[0] + _bf(xb[1]) * _GRAY_W[1]) + _bf(xb[2]) * _GRAY_W[2]

    g2 = _gauss2d_bf16()
    gp = _reflect_pad(_bf(gray), 2, 2)
    T = [gp[:, j:j + W] for j in range(_KSIZE)]
    bl = g2[0][0] * T[0][0:H]
    for i in range(_KSIZE):
        for j in range(_KSIZE):
            if i == 0 and j == 0:
                continue
            bl = bl + g2[i][j] * T[j][i:i + H]

    bp = _reflect_pad(_bf(bl), 1, 1)
    S0 = bp[:, 0:W]
    S1 = bp[:, 1:W + 1]
    S2 = bp[:, 2:W + 2]
    p00 = S0[0:H];     p01 = S1[0:H];     p02 = S2[0:H]
    p10 = S0[1:H + 1];                    p12 = S2[1:H + 1]
    p20 = S0[2:H + 2]; p21 = S1[2:H + 2]; p22 = S2[2:H + 2]
    gx = -p00 + p02 - 2.0 * p10 + 2.0 * p12 - p20 + p22
    gy = -p00 - 2.0 * p01 - p02 + p20 + 2.0 * p21 + p22

    mag = jnp.hypot(gx, gy)
    m = jnp.max(mag, keepdims=True)
    magn = mag / (m + 1e-12)

    ax = jnp.abs(gx)
    ay = jnp.abs(gy)
    pos = ((gx > 0) & (gy > 0)) | ((gx < 0) & (gy < 0))
    neg = ((gx > 0) & (gy < 0)) | ((gx < 0) & (gy > 0))
    t1ax = _T1 * ax
    t2ax = _T2 * ax
    b1 = pos & (ay >= t1ax) & (ay < t2ax)
    b2 = (pos & (ay >= t2ax)) | (~pos & (ay > t2ax))
    b3 = neg & (ay > t1ax) & (ay <= t2ax)

    mp = _zero_pad1(magn)
    M0 = mp[:, 0:W]
    M1 = mp[:, 1:W + 1]
    M2 = mp[:, 2:W + 2]
    E = M2[1:H + 1];  W_ = M0[1:H + 1]
    S = M1[2:H + 2];  N = M1[0:H]
    SW = M0[2:H + 2]; NE = M2[0:H]
    NW = M0[0:H];     SE = M2[2:H + 2]
    q = jnp.where(b1, SW, jnp.where(b2, S, jnp.where(b3, NW, E)))
    r = jnp.where(b1, NE, jnp.where(b2, N, jnp.where(b3, SE, W_)))
    keep = (magn >= q) & (magn >= r)

    row = lax.broadcasted_iota(jnp.int32, (H, W), 0)
    col = lax.broadcasted_iota(jnp.int32, (H, W), 1)
    interior = (row >= 1) & (row < H - 1) & (col >= 1) & (col < W - 1)
    nms = jnp.where(keep & interior, magn, 0.0)

    strong = (nms >= _HIGH_T).astype(jnp.int32)
    weak = ((nms >= _LOW_T) & (nms < _HIGH_T)).astype(jnp.int32)
    return strong, weak


def _pack_rows(mask_i32, H, W):
    rowk = lax.broadcasted_iota(jnp.int32, (H, W), 0) & 31
    sh = mask_i32 << rowk
    r = H
    while r > H // 32:
        sh = jnp.sum(sh.reshape(r // 2, 2, W), axis=1)
        r //= 2
    return sh


def _unpack_rows(packed, H, W):
    rep = jnp.broadcast_to(packed.reshape(H // 32, 1, W), (H // 32, 32, W))
    rep = rep.reshape(H, W)
    rowk = lax.broadcasted_iota(jnp.int32, (H, W), 0) & 31
    bits = (rep >> rowk) & 1
    return bits.astype(jnp.float32)


def _hysteresis_loop(s_ref, w_ref, G, Hp, W):
    def dilate_once(s, w):
        v = s | (s << 1) | _lsr(s, 1)
        zr = jnp.zeros((G, 1, W), jnp.int32)
        up = jnp.concatenate([zr, s[:, 0:Hp - 1]], axis=1)
        dn = jnp.concatenate([s[:, 1:Hp], zr], axis=1)
        v = v | (_lsr(up, 31) & 1) | ((dn & 1) << 31)
        zc = jnp.zeros((G, Hp, 1), jnp.int32)
        hp = jnp.concatenate([zc, v, zc], axis=2)
        h = v | hp[:, :, 0:W] | hp[:, :, 2:W + 2]
        return s | (w & h)

    def body(carry):
        _, it = carry
        s = s_ref[...]
        w = w_ref[...]
        new = dilate_once(dilate_once(s, w), w)
        s_ref[...] = new
        changed = jnp.max(new ^ s) > 0
        return changed, it + 2

    def cond(carry):
        changed, it = carry
        return changed & (it < _MAX_HYST_ITERS)

    lax.while_loop(cond, body, (jnp.bool_(True), jnp.int32(0)))


def _lsr(x, k):
    return lax.shift_right_logical(x, jnp.int32(k))


_G = 2


def _canny_body(x_ref, o_ref, s_ref, w_ref):
    H, W = o_ref.shape[2], o_ref.shape[3]
    for g in range(_G):
        strong, weak = _nms_stage(x_ref[g], H, W)
        s_ref[g] = _pack_rows(strong, H, W)
        w_ref[g] = _pack_rows(weak, H, W)
    _hysteresis_loop(s_ref, w_ref, _G, H // 32, W)
    for g in range(_G):
        o_ref[g, 0] = _unpack_rows(s_ref[g], H, W)


@jax.jit
def kernel(x):
    B, C, H, W = x.shape
    out = pl.pallas_call(
        _canny_body,
        grid=(B // _G,),
        in_specs=[pl.BlockSpec((_G, C, H, W), lambda i: (i, 0, 0, 0))],
        out_specs=pl.BlockSpec((_G, 1, H, W), lambda i: (i, 0, 0, 0)),
        out_shape=jax.ShapeDtypeStruct((B, 1, H, W), jnp.float32),
        scratch_shapes=[
            pltpu.VMEM((_G, H // 32, W), jnp.int32),
            pltpu.VMEM((_G, H // 32, W), jnp.int32),
        ],
        compiler_params=pltpu.CompilerParams(
            dimension_semantics=("parallel",),
            vmem_limit_bytes=64 * 1024 * 1024,
        ),
    )(x)
    return out


# speedup vs baseline: 1.1760x; 1.1760x over previous
import math

import jax
import jax.numpy as jnp
from jax import lax
from jax.experimental import pallas as pl
from jax.experimental.pallas import tpu as pltpu
import ml_dtypes
import numpy as np

_KSIZE = 5
_SIGMA = 1.4
_LOW_T = 0.1
_HIGH_T = 0.3
_MAX_HYST_ITERS = 256
_T1 = math.tan(math.radians(22.5))
_T2 = math.tan(math.radians(67.5))


def _bf16_round(v):
    return float(np.float32(np.float32(v).astype(ml_dtypes.bfloat16)))


def _gauss2d_bf16():
    c = (_KSIZE - 1) / 2.0
    g = np.exp(-((np.arange(_KSIZE) - c) ** 2) / (2.0 * _SIGMA * _SIGMA))
    g = g / g.sum()
    k = np.outer(g, g).astype(np.float32)
    return [[_bf16_round(k[i, j]) for j in range(_KSIZE)] for i in range(_KSIZE)]


_GRAY_W = [_bf16_round(v) for v in (0.299, 0.587, 0.114)]


def _bf(t):
    return t.astype(jnp.bfloat16).astype(jnp.float32)


def _reflect_pad(a, ph, pw):
    if ph:
        top = [a[ph - i:ph + 1 - i] for i in range(ph)]
        bot = [a[a.shape[0] - 2 - i:a.shape[0] - 1 - i] for i in range(ph)]
        a = jnp.concatenate(top + [a] + bot, axis=0)
    if pw:
        left = [a[:, pw - i:pw + 1 - i] for i in range(pw)]
        right = [a[:, a.shape[1] - 2 - i:a.shape[1] - 1 - i] for i in range(pw)]
        a = jnp.concatenate(left + [a] + right, axis=1)
    return a


def _zero_pad1(a):
    H, W = a.shape
    z_row = jnp.zeros((1, W), a.dtype)
    a = jnp.concatenate([z_row, a, z_row], axis=0)
    z_col = jnp.zeros((H + 2, 1), a.dtype)
    return jnp.concatenate([z_col, a, z_col], axis=1)


def _nms_stage(xb, H, W):
    gray = (_bf(xb[0]) * _GRAY_W[0] + _bf(xb[1]) * _GRAY_W[1]) + _bf(xb[2]) * _GRAY_W[2]

    g2 = _gauss2d_bf16()
    gp = _reflect_pad(_bf(gray), 2, 2)
    T = [gp[:, j:j + W] for j in range(_KSIZE)]
    bl = g2[0][0] * T[0][0:H]
    for i in range(_KSIZE):
        for j in range(_KSIZE):
            if i == 0 and j == 0:
                continue
            bl = bl + g2[i][j] * T[j][i:i + H]

    bp = _reflect_pad(_bf(bl), 1, 1)
    S0 = bp[:, 0:W]
    S1 = bp[:, 1:W + 1]
    S2 = bp[:, 2:W + 2]
    p00 = S0[0:H];     p01 = S1[0:H];     p02 = S2[0:H]
    p10 = S0[1:H + 1];                    p12 = S2[1:H + 1]
    p20 = S0[2:H + 2]; p21 = S1[2:H + 2]; p22 = S2[2:H + 2]
    gx = -p00 + p02 - 2.0 * p10 + 2.0 * p12 - p20 + p22
    gy = -p00 - 2.0 * p01 - p02 + p20 + 2.0 * p21 + p22

    mag = jnp.hypot(gx, gy)
    m = jnp.max(mag, keepdims=True)
    magn = mag / (m + 1e-12)

    ax = jnp.abs(gx)
    ay = jnp.abs(gy)
    pos = ((gx > 0) & (gy > 0)) | ((gx < 0) & (gy < 0))
    neg = ((gx > 0) & (gy < 0)) | ((gx < 0) & (gy > 0))
    t1ax = _T1 * ax
    t2ax = _T2 * ax
    b1 = pos & (ay >= t1ax) & (ay < t2ax)
    b2 = (pos & (ay >= t2ax)) | (~pos & (ay > t2ax))
    b3 = neg & (ay > t1ax) & (ay <= t2ax)

    mp = _zero_pad1(magn)
    M0 = mp[:, 0:W]
    M1 = mp[:, 1:W + 1]
    M2 = mp[:, 2:W + 2]
    E = M2[1:H + 1];  W_ = M0[1:H + 1]
    S = M1[2:H + 2];  N = M1[0:H]
    SW = M0[2:H + 2]; NE = M2[0:H]
    NW = M0[0:H];     SE = M2[2:H + 2]
    q = jnp.where(b1, SW, jnp.where(b2, S, jnp.where(b3, NW, E)))
    r = jnp.where(b1, NE, jnp.where(b2, N, jnp.where(b3, SE, W_)))
    keep = (magn >= q) & (magn >= r)

    row = lax.broadcasted_iota(jnp.int32, (H, W), 0)
    col = lax.broadcasted_iota(jnp.int32, (H, W), 1)
    interior = (row >= 1) & (row < H - 1) & (col >= 1) & (col < W - 1)
    nms = jnp.where(keep & interior, magn, 0.0)

    strong = (nms >= _HIGH_T).astype(jnp.int32)
    weak = ((nms >= _LOW_T) & (nms < _HIGH_T)).astype(jnp.int32)
    return strong, weak


def _pack_rows(mask_i32, H, W):
    rowk = lax.broadcasted_iota(jnp.int32, (H, W), 0) & 31
    sh = mask_i32 << rowk
    r = H
    while r > H // 32:
        sh = jnp.sum(sh.reshape(r // 2, 2, W), axis=1)
        r //= 2
    return sh


def _unpack_rows(packed, H, W):
    rep = jnp.broadcast_to(packed.reshape(H // 32, 1, W), (H // 32, 32, W))
    rep = rep.reshape(H, W)
    rowk = lax.broadcasted_iota(jnp.int32, (H, W), 0) & 31
    bits = (rep >> rowk) & 1
    return bits.astype(jnp.float32)


def _hysteresis_loop(s_ref, w_ref, G, Hp, W):
    def dilate_once(s, w):
        v = s | (s << 1) | _lsr(s, 1)
        zr = jnp.zeros((G, 1, W), jnp.int32)
        up = jnp.concatenate([zr, s[:, 0:Hp - 1]], axis=1)
        dn = jnp.concatenate([s[:, 1:Hp], zr], axis=1)
        v = v | (_lsr(up, 31) & 1) | ((dn & 1) << 31)
        zc = jnp.zeros((G, Hp, 1), jnp.int32)
        hp = jnp.concatenate([zc, v, zc], axis=2)
        h = v | hp[:, :, 0:W] | hp[:, :, 2:W + 2]
        return s | (w & h)

    def body(carry):
        _, it = carry
        s = s_ref[...]
        w = w_ref[...]
        new = dilate_once(dilate_once(s, w), w)
        s_ref[...] = new
        changed = jnp.max(new ^ s) > 0
        return changed, it + 2

    def cond(carry):
        changed, it = carry
        return changed & (it < _MAX_HYST_ITERS)

    lax.while_loop(cond, body, (jnp.bool_(True), jnp.int32(0)))


def _lsr(x, k):
    return lax.shift_right_logical(x, jnp.int32(k))


_G = 1


def _canny_body(x_ref, o_ref, s_ref, w_ref):
    H, W = o_ref.shape[2], o_ref.shape[3]
    for g in range(_G):
        strong, weak = _nms_stage(x_ref[g], H, W)
        s_ref[g] = _pack_rows(strong, H, W)
        w_ref[g] = _pack_rows(weak, H, W)
    _hysteresis_loop(s_ref, w_ref, _G, H // 32, W)
    for g in range(_G):
        o_ref[g, 0] = _unpack_rows(s_ref[g], H, W)


@jax.jit
def kernel(x):
    B, C, H, W = x.shape
    out = pl.pallas_call(
        _canny_body,
        grid=(B // _G,),
        in_specs=[pl.BlockSpec((_G, C, H, W), lambda i: (i, 0, 0, 0))],
        out_specs=pl.BlockSpec((_G, 1, H, W), lambda i: (i, 0, 0, 0)),
        out_shape=jax.ShapeDtypeStruct((B, 1, H, W), jnp.float32),
        scratch_shapes=[
            pltpu.VMEM((_G, H // 32, W), jnp.int32),
            pltpu.VMEM((_G, H // 32, W), jnp.int32),
        ],
        compiler_params=pltpu.CompilerParams(
            dimension_semantics=("parallel",),
            vmem_limit_bytes=64 * 1024 * 1024,
        ),
    )(x)
    return out


# speedup vs baseline: 1.3140x; 1.1174x over previous
import math

import jax
import jax.numpy as jnp
from jax import lax
from jax.experimental import pallas as pl
from jax.experimental.pallas import tpu as pltpu
import ml_dtypes
import numpy as np

_KSIZE = 5
_SIGMA = 1.4
_LOW_T = 0.1
_HIGH_T = 0.3
_MAX_HYST_ITERS = 256
_T1 = math.tan(math.radians(22.5))
_T2 = math.tan(math.radians(67.5))


def _bf16_round(v):
    return float(np.float32(np.float32(v).astype(ml_dtypes.bfloat16)))


def _gauss2d_bf16():
    c = (_KSIZE - 1) / 2.0
    g = np.exp(-((np.arange(_KSIZE) - c) ** 2) / (2.0 * _SIGMA * _SIGMA))
    g = g / g.sum()
    k = np.outer(g, g).astype(np.float32)
    return [[_bf16_round(k[i, j]) for j in range(_KSIZE)] for i in range(_KSIZE)]


_GRAY_W = [_bf16_round(v) for v in (0.299, 0.587, 0.114)]


def _bf(t):
    return t.astype(jnp.bfloat16).astype(jnp.float32)


def _reflect_pad(a, ph, pw):
    if ph:
        top = [a[ph - i:ph + 1 - i] for i in range(ph)]
        bot = [a[a.shape[0] - 2 - i:a.shape[0] - 1 - i] for i in range(ph)]
        a = jnp.concatenate(top + [a] + bot, axis=0)
    if pw:
        left = [a[:, pw - i:pw + 1 - i] for i in range(pw)]
        right = [a[:, a.shape[1] - 2 - i:a.shape[1] - 1 - i] for i in range(pw)]
        a = jnp.concatenate(left + [a] + right, axis=1)
    return a


def _zero_pad1(a):
    H, W = a.shape
    z_row = jnp.zeros((1, W), a.dtype)
    a = jnp.concatenate([z_row, a, z_row], axis=0)
    z_col = jnp.zeros((H + 2, 1), a.dtype)
    return jnp.concatenate([z_col, a, z_col], axis=1)


def _nms_stage(xb, H, W):
    gray = (_bf(xb[0]) * _GRAY_W[0] + _bf(xb[1]) * _GRAY_W[1]) + _bf(xb[2]) * _GRAY_W[2]

    g2 = _gauss2d_bf16()
    gp = _reflect_pad(_bf(gray), 2, 2)
    T = [gp[:, j:j + W] for j in range(_KSIZE)]
    bl = g2[0][0] * T[0][0:H]
    for i in range(_KSIZE):
        for j in range(_KSIZE):
            if i == 0 and j == 0:
                continue
            bl = bl + g2[i][j] * T[j][i:i + H]

    bp = _reflect_pad(_bf(bl), 1, 1)
    S0 = bp[:, 0:W]
    S1 = bp[:, 1:W + 1]
    S2 = bp[:, 2:W + 2]
    p00 = S0[0:H];     p01 = S1[0:H];     p02 = S2[0:H]
    p10 = S0[1:H + 1];                    p12 = S2[1:H + 1]
    p20 = S0[2:H + 2]; p21 = S1[2:H + 2]; p22 = S2[2:H + 2]
    gx = -p00 + p02 - 2.0 * p10 + 2.0 * p12 - p20 + p22
    gy = -p00 - 2.0 * p01 - p02 + p20 + 2.0 * p21 + p22

    mag = jnp.hypot(gx, gy)
    m = jnp.max(mag, keepdims=True)
    magn = mag / (m + 1e-12)

    ax = jnp.abs(gx)
    ay = jnp.abs(gy)
    pos = ((gx > 0) & (gy > 0)) | ((gx < 0) & (gy < 0))
    neg = ((gx > 0) & (gy < 0)) | ((gx < 0) & (gy > 0))
    t1ax = _T1 * ax
    t2ax = _T2 * ax
    b1 = pos & (ay >= t1ax) & (ay < t2ax)
    b2 = (pos & (ay >= t2ax)) | (~pos & (ay > t2ax))
    b3 = neg & (ay > t1ax) & (ay <= t2ax)

    mp = _zero_pad1(magn)
    M0 = mp[:, 0:W]
    M1 = mp[:, 1:W + 1]
    M2 = mp[:, 2:W + 2]
    E = M2[1:H + 1];  W_ = M0[1:H + 1]
    S = M1[2:H + 2];  N = M1[0:H]
    SW = M0[2:H + 2]; NE = M2[0:H]
    NW = M0[0:H];     SE = M2[2:H + 2]
    q = jnp.where(b1, SW, jnp.where(b2, S, jnp.where(b3, NW, E)))
    r = jnp.where(b1, NE, jnp.where(b2, N, jnp.where(b3, SE, W_)))
    keep = (magn >= q) & (magn >= r)

    row = lax.broadcasted_iota(jnp.int32, (H, W), 0)
    col = lax.broadcasted_iota(jnp.int32, (H, W), 1)
    interior = (row >= 1) & (row < H - 1) & (col >= 1) & (col < W - 1)
    nms = jnp.where(keep & interior, magn, 0.0)

    strong = jnp.where(nms >= _HIGH_T, 1.0, 0.0)
    weak = jnp.where((nms >= _LOW_T) & (nms < _HIGH_T), 1.0, 0.0)
    return strong, weak


def _pack_rows(mask_f32, H, W):
    rows16 = H // 16
    cj = lax.broadcasted_iota(jnp.int32, (rows16, H), 1)
    rj = lax.broadcasted_iota(jnp.int32, (rows16, H), 0)
    pmat = jnp.where((cj >> 4) == rj, (1 << (cj & 15)), 0).astype(jnp.float32)
    p16 = jnp.dot(pmat.astype(jnp.bfloat16), mask_f32.astype(jnp.bfloat16),
                  preferred_element_type=jnp.float32)
    pi = p16.astype(jnp.int32).reshape(H // 32, 2, W)
    return pi[:, 0, :] | (pi[:, 1, :] << 16)


def _unpack_rows(packed, H, W):
    rep = jnp.broadcast_to(packed.reshape(H // 32, 1, W), (H // 32, 32, W))
    rep = rep.reshape(H, W)
    rowk = lax.broadcasted_iota(jnp.int32, (H, W), 0) & 31
    bits = (rep >> rowk) & 1
    return bits.astype(jnp.float32)


def _hysteresis_loop(s_ref, w_ref, G, Hp, W):
    def dilate_once(s, w):
        v = s | (s << 1) | _lsr(s, 1)
        zr = jnp.zeros((G, 1, W), jnp.int32)
        up = jnp.concatenate([zr, s[:, 0:Hp - 1]], axis=1)
        dn = jnp.concatenate([s[:, 1:Hp], zr], axis=1)
        v = v | (_lsr(up, 31) & 1) | ((dn & 1) << 31)
        zc = jnp.zeros((G, Hp, 1), jnp.int32)
        hp = jnp.concatenate([zc, v, zc], axis=2)
        h = v | hp[:, :, 0:W] | hp[:, :, 2:W + 2]
        return s | (w & h)

    def body(carry):
        _, it = carry
        s = s_ref[...]
        w = w_ref[...]
        new = dilate_once(dilate_once(s, w), w)
        s_ref[...] = new
        changed = jnp.max(new ^ s) > 0
        return changed, it + 2

    def cond(carry):
        changed, it = carry
        return changed & (it < _MAX_HYST_ITERS)

    lax.while_loop(cond, body, (jnp.bool_(True), jnp.int32(0)))


def _lsr(x, k):
    return lax.shift_right_logical(x, jnp.int32(k))


_G = 1


def _canny_body(x_ref, o_ref, s_ref, w_ref):
    H, W = o_ref.shape[2], o_ref.shape[3]
    for g in range(_G):
        strong, weak = _nms_stage(x_ref[g], H, W)
        s_ref[g] = _pack_rows(strong, H, W)
        w_ref[g] = _pack_rows(weak, H, W)
    _hysteresis_loop(s_ref, w_ref, _G, H // 32, W)
    for g in range(_G):
        o_ref[g, 0] = _unpack_rows(s_ref[g], H, W)


@jax.jit
def kernel(x):
    B, C, H, W = x.shape
    out = pl.pallas_call(
        _canny_body,
        grid=(B // _G,),
        in_specs=[pl.BlockSpec((_G, C, H, W), lambda i: (i, 0, 0, 0))],
        out_specs=pl.BlockSpec((_G, 1, H, W), lambda i: (i, 0, 0, 0)),
        out_shape=jax.ShapeDtypeStruct((B, 1, H, W), jnp.float32),
        scratch_shapes=[
            pltpu.VMEM((_G, H // 32, W), jnp.int32),
            pltpu.VMEM((_G, H // 32, W), jnp.int32),
        ],
        compiler_params=pltpu.CompilerParams(
            dimension_semantics=("parallel",),
            vmem_limit_bytes=64 * 1024 * 1024,
        ),
    )(x)
    return out


# speedup vs baseline: 1.9604x; 1.4919x over previous
import math

import jax
import jax.numpy as jnp
from jax import lax
from jax.experimental import pallas as pl
from jax.experimental.pallas import tpu as pltpu
import ml_dtypes
import numpy as np

_KSIZE = 5
_SIGMA = 1.4
_LOW_T = 0.1
_HIGH_T = 0.3
_MAX_HYST_ITERS = 256
_T1 = math.tan(math.radians(22.5))
_T2 = math.tan(math.radians(67.5))


def _bf16_round(v):
    return float(np.float32(np.float32(v).astype(ml_dtypes.bfloat16)))


def _gauss2d_bf16():
    c = (_KSIZE - 1) / 2.0
    g = np.exp(-((np.arange(_KSIZE) - c) ** 2) / (2.0 * _SIGMA * _SIGMA))
    g = g / g.sum()
    k = np.outer(g, g).astype(np.float32)
    return [[_bf16_round(k[i, j]) for j in range(_KSIZE)] for i in range(_KSIZE)]


_GRAY_W = [_bf16_round(v) for v in (0.299, 0.587, 0.114)]


def _bf(t):
    return t.astype(jnp.bfloat16).astype(jnp.float32)


def _reflect_pad(a, ph, pw):
    if ph:
        top = [a[ph - i:ph + 1 - i] for i in range(ph)]
        bot = [a[a.shape[0] - 2 - i:a.shape[0] - 1 - i] for i in range(ph)]
        a = jnp.concatenate(top + [a] + bot, axis=0)
    if pw:
        left = [a[:, pw - i:pw + 1 - i] for i in range(pw)]
        right = [a[:, a.shape[1] - 2 - i:a.shape[1] - 1 - i] for i in range(pw)]
        a = jnp.concatenate(left + [a] + right, axis=1)
    return a


def _zero_pad1(a):
    H, W = a.shape
    z_row = jnp.zeros((1, W), a.dtype)
    a = jnp.concatenate([z_row, a, z_row], axis=0)
    z_col = jnp.zeros((H + 2, 1), a.dtype)
    return jnp.concatenate([z_col, a, z_col], axis=1)


_KPAD = 640


def _nms_stage(xb, bands_ref, H, W):
    gray = (_bf(xb[0]) * _GRAY_W[0] + _bf(xb[1]) * _GRAY_W[1]) + _bf(xb[2]) * _GRAY_W[2]

    gp = _reflect_pad(_bf(gray), 2, 2)
    gpz = jnp.concatenate(
        [gp, jnp.zeros((H + 4, _KPAD - (W + 4)), jnp.float32)], axis=1)
    gpb = gpz.astype(jnp.bfloat16)
    M = [jnp.dot(gpb, bands_ref[i], preferred_element_type=jnp.float32)
         for i in range(_KSIZE)]
    bl = M[0][0:H]
    for i in range(1, _KSIZE):
        bl = bl + M[i][i:i + H]

    bp = _reflect_pad(_bf(bl), 1, 1)
    S0 = bp[:, 0:W]
    S1 = bp[:, 1:W + 1]
    S2 = bp[:, 2:W + 2]
    p00 = S0[0:H];     p01 = S1[0:H];     p02 = S2[0:H]
    p10 = S0[1:H + 1];                    p12 = S2[1:H + 1]
    p20 = S0[2:H + 2]; p21 = S1[2:H + 2]; p22 = S2[2:H + 2]
    gx = -p00 + p02 - 2.0 * p10 + 2.0 * p12 - p20 + p22
    gy = -p00 - 2.0 * p01 - p02 + p20 + 2.0 * p21 + p22

    mag = jnp.hypot(gx, gy)
    m = jnp.max(mag, keepdims=True)
    magn = mag / (m + 1e-12)

    ax = jnp.abs(gx)
    ay = jnp.abs(gy)
    pos = ((gx > 0) & (gy > 0)) | ((gx < 0) & (gy < 0))
    neg = ((gx > 0) & (gy < 0)) | ((gx < 0) & (gy > 0))
    t1ax = _T1 * ax
    t2ax = _T2 * ax
    b1 = pos & (ay >= t1ax) & (ay < t2ax)
    b2 = (pos & (ay >= t2ax)) | (~pos & (ay > t2ax))
    b3 = neg & (ay > t1ax) & (ay <= t2ax)

    mp = _zero_pad1(magn)
    M0 = mp[:, 0:W]
    M1 = mp[:, 1:W + 1]
    M2 = mp[:, 2:W + 2]
    E = M2[1:H + 1];  W_ = M0[1:H + 1]
    S = M1[2:H + 2];  N = M1[0:H]
    SW = M0[2:H + 2]; NE = M2[0:H]
    NW = M0[0:H];     SE = M2[2:H + 2]
    q = jnp.where(b1, SW, jnp.where(b2, S, jnp.where(b3, NW, E)))
    r = jnp.where(b1, NE, jnp.where(b2, N, jnp.where(b3, SE, W_)))
    keep = (magn >= q) & (magn >= r)

    row = lax.broadcasted_iota(jnp.int32, (H, W), 0)
    col = lax.broadcasted_iota(jnp.int32, (H, W), 1)
    interior = (row >= 1) & (row < H - 1) & (col >= 1) & (col < W - 1)
    nms = jnp.where(keep & interior, magn, 0.0)

    strong = jnp.where(nms >= _HIGH_T, 1.0, 0.0)
    weak = jnp.where((nms >= _LOW_T) & (nms < _HIGH_T), 1.0, 0.0)
    return strong, weak


def _pack_rows(mask_f32, H, W):
    rows16 = H // 16
    cj = lax.broadcasted_iota(jnp.int32, (rows16, H), 1)
    rj = lax.broadcasted_iota(jnp.int32, (rows16, H), 0)
    pmat = jnp.where((cj >> 4) == rj, (1 << (cj & 15)), 0).astype(jnp.float32)
    p16 = jnp.dot(pmat.astype(jnp.bfloat16), mask_f32.astype(jnp.bfloat16),
                  preferred_element_type=jnp.float32)
    pi = p16.astype(jnp.int32).reshape(H // 32, 2, W)
    return pi[:, 0, :] | (pi[:, 1, :] << 16)


def _unpack_rows(packed, H, W):
    rep = jnp.broadcast_to(packed.reshape(H // 32, 1, W), (H // 32, 32, W))
    rep = rep.reshape(H, W)
    rowk = lax.broadcasted_iota(jnp.int32, (H, W), 0) & 31
    bits = (rep >> rowk) & 1
    return bits.astype(jnp.float32)


def _hysteresis_loop(s_ref, w_ref, G, Hp, W):
    def dilate_once(s, w):
        v = s | (s << 1) | _lsr(s, 1)
        zr = jnp.zeros((G, 1, W), jnp.int32)
        up = jnp.concatenate([zr, s[:, 0:Hp - 1]], axis=1)
        dn = jnp.concatenate([s[:, 1:Hp], zr], axis=1)
        v = v | (_lsr(up, 31) & 1) | ((dn & 1) << 31)
        zc = jnp.zeros((G, Hp, 1), jnp.int32)
        hp = jnp.concatenate([zc, v, zc], axis=2)
        h = v | hp[:, :, 0:W] | hp[:, :, 2:W + 2]
        return s | (w & h)

    def body(carry):
        _, it = carry
        s = s_ref[...]
        w = w_ref[...]
        new = dilate_once(dilate_once(s, w), w)
        s_ref[...] = new
        changed = jnp.max(new ^ s) > 0
        return changed, it + 2

    def cond(carry):
        changed, it = carry
        return changed & (it < _MAX_HYST_ITERS)

    lax.while_loop(cond, body, (jnp.bool_(True), jnp.int32(0)))


def _lsr(x, k):
    return lax.shift_right_logical(x, jnp.int32(k))


_G = 1


def _canny_body(x_ref, bands_ref, o_ref, s_ref, w_ref):
    H, W = o_ref.shape[2], o_ref.shape[3]
    for g in range(_G):
        strong, weak = _nms_stage(x_ref[g], bands_ref, H, W)
        s_ref[g] = _pack_rows(strong, H, W)
        w_ref[g] = _pack_rows(weak, H, W)
    _hysteresis_loop(s_ref, w_ref, _G, H // 32, W)
    for g in range(_G):
        o_ref[g, 0] = _unpack_rows(s_ref[g], H, W)


def _blur_bands(W):
    g2 = np.array(_gauss2d_bf16(), np.float32)
    bands = np.zeros((_KSIZE, _KPAD, W), np.float32)
    for i in range(_KSIZE):
        for j in range(_KSIZE):
            for c in range(W):
                bands[i, c + j, c] = g2[i, j]
    return bands


@jax.jit
def kernel(x):
    B, C, H, W = x.shape
    bands = jnp.asarray(_blur_bands(W), dtype=jnp.bfloat16)
    out = pl.pallas_call(
        _canny_body,
        grid=(B // _G,),
        in_specs=[pl.BlockSpec((_G, C, H, W), lambda i: (i, 0, 0, 0)),
                  pl.BlockSpec((_KSIZE, _KPAD, W), lambda i: (0, 0, 0))],
        out_specs=pl.BlockSpec((_G, 1, H, W), lambda i: (i, 0, 0, 0)),
        out_shape=jax.ShapeDtypeStruct((B, 1, H, W), jnp.float32),
        scratch_shapes=[
            pltpu.VMEM((_G, H // 32, W), jnp.int32),
            pltpu.VMEM((_G, H // 32, W), jnp.int32),
        ],
        compiler_params=pltpu.CompilerParams(
            dimension_semantics=("parallel",),
            vmem_limit_bytes=64 * 1024 * 1024,
        ),
    )(x, bands)
    return out


# speedup vs baseline: 2.2210x; 1.1329x over previous
import math

import jax
import jax.numpy as jnp
from jax import lax
from jax.experimental import pallas as pl
from jax.experimental.pallas import tpu as pltpu
import ml_dtypes
import numpy as np

_KSIZE = 5
_SIGMA = 1.4
_LOW_T = 0.1
_HIGH_T = 0.3
_MAX_HYST_ITERS = 256
_T1 = math.tan(math.radians(22.5))
_T2 = math.tan(math.radians(67.5))


def _bf16_round(v):
    return float(np.float32(np.float32(v).astype(ml_dtypes.bfloat16)))


def _gauss2d_bf16():
    c = (_KSIZE - 1) / 2.0
    g = np.exp(-((np.arange(_KSIZE) - c) ** 2) / (2.0 * _SIGMA * _SIGMA))
    g = g / g.sum()
    k = np.outer(g, g).astype(np.float32)
    return [[_bf16_round(k[i, j]) for j in range(_KSIZE)] for i in range(_KSIZE)]


_GRAY_W = [_bf16_round(v) for v in (0.299, 0.587, 0.114)]


def _bf(t):
    return t.astype(jnp.bfloat16).astype(jnp.float32)


def _reflect_pad(a, ph, pw):
    if ph:
        top = [a[ph - i:ph + 1 - i] for i in range(ph)]
        bot = [a[a.shape[0] - 2 - i:a.shape[0] - 1 - i] for i in range(ph)]
        a = jnp.concatenate(top + [a] + bot, axis=0)
    if pw:
        left = [a[:, pw - i:pw + 1 - i] for i in range(pw)]
        right = [a[:, a.shape[1] - 2 - i:a.shape[1] - 1 - i] for i in range(pw)]
        a = jnp.concatenate(left + [a] + right, axis=1)
    return a


def _zero_pad1(a):
    H, W = a.shape
    z_row = jnp.zeros((1, W), a.dtype)
    a = jnp.concatenate([z_row, a, z_row], axis=0)
    z_col = jnp.zeros((H + 2, 1), a.dtype)
    return jnp.concatenate([z_col, a, z_col], axis=1)


_KPAD = 640


def _nms_stage(xb, bands_ref, H, W):
    gray = (_bf(xb[0]) * _GRAY_W[0] + _bf(xb[1]) * _GRAY_W[1]) + _bf(xb[2]) * _GRAY_W[2]

    gp = _reflect_pad(_bf(gray), 2, 2)
    gpz = jnp.concatenate(
        [gp, jnp.zeros((H + 4, _KPAD - (W + 4)), jnp.float32)], axis=1)
    gpb = gpz.astype(jnp.bfloat16)
    M = [jnp.dot(gpb, bands_ref[i], preferred_element_type=jnp.float32)
         for i in range(_KSIZE)]
    bl = M[0][0:H]
    for i in range(1, _KSIZE):
        bl = bl + M[i][i:i + H]

    bp = _reflect_pad(_bf(bl), 1, 1)
    S0 = bp[:, 0:W]
    S1 = bp[:, 1:W + 1]
    S2 = bp[:, 2:W + 2]
    p00 = S0[0:H];     p01 = S1[0:H];     p02 = S2[0:H]
    p10 = S0[1:H + 1];                    p12 = S2[1:H + 1]
    p20 = S0[2:H + 2]; p21 = S1[2:H + 2]; p22 = S2[2:H + 2]
    gx = -p00 + p02 - 2.0 * p10 + 2.0 * p12 - p20 + p22
    gy = -p00 - 2.0 * p01 - p02 + p20 + 2.0 * p21 + p22

    mag = jnp.hypot(gx, gy)
    m = jnp.max(mag, keepdims=True)
    magn = mag / (m + 1e-12)

    ax = jnp.abs(gx)
    ay = jnp.abs(gy)
    pos = ((gx > 0) & (gy > 0)) | ((gx < 0) & (gy < 0))
    neg = ((gx > 0) & (gy < 0)) | ((gx < 0) & (gy > 0))
    t1ax = _T1 * ax
    t2ax = _T2 * ax
    b1 = pos & (ay >= t1ax) & (ay < t2ax)
    b2 = (pos & (ay >= t2ax)) | (~pos & (ay > t2ax))
    b3 = neg & (ay > t1ax) & (ay <= t2ax)

    mp = _zero_pad1(magn)
    M0 = mp[:, 0:W]
    M1 = mp[:, 1:W + 1]
    M2 = mp[:, 2:W + 2]
    E = M2[1:H + 1];  W_ = M0[1:H + 1]
    S = M1[2:H + 2];  N = M1[0:H]
    SW = M0[2:H + 2]; NE = M2[0:H]
    NW = M0[0:H];     SE = M2[2:H + 2]
    q = jnp.where(b1, SW, jnp.where(b2, S, jnp.where(b3, NW, E)))
    r = jnp.where(b1, NE, jnp.where(b2, N, jnp.where(b3, SE, W_)))
    keep = (magn >= q) & (magn >= r)

    row = lax.broadcasted_iota(jnp.int32, (H, W), 0)
    col = lax.broadcasted_iota(jnp.int32, (H, W), 1)
    interior = (row >= 1) & (row < H - 1) & (col >= 1) & (col < W - 1)
    nms = jnp.where(keep & interior, magn, 0.0)

    strong = jnp.where(nms >= _HIGH_T, 1.0, 0.0)
    weak = jnp.where((nms >= _LOW_T) & (nms < _HIGH_T), 1.0, 0.0)
    return strong, weak


def _pack_rows(mask_f32, H, W):
    rows16 = H // 16
    cj = lax.broadcasted_iota(jnp.int32, (rows16, H), 1)
    rj = lax.broadcasted_iota(jnp.int32, (rows16, H), 0)
    pmat = jnp.where((cj >> 4) == rj, (1 << (cj & 15)), 0).astype(jnp.float32)
    p16 = jnp.dot(pmat.astype(jnp.bfloat16), mask_f32.astype(jnp.bfloat16),
                  preferred_element_type=jnp.float32)
    pi = p16.astype(jnp.int32).reshape(H // 32, 2, W)
    return pi[:, 0, :] | (pi[:, 1, :] << 16)


def _unpack_rows(packed, H, W):
    rep = jnp.broadcast_to(packed.reshape(H // 32, 1, W), (H // 32, 32, W))
    rep = rep.reshape(H, W)
    rowk = lax.broadcasted_iota(jnp.int32, (H, W), 0) & 31
    bits = (rep >> rowk) & 1
    return bits.astype(jnp.float32)


def _hysteresis_loop(s_ref, w_ref, G, Hp, W):
    def dilate_once(s, w):
        v = s | (s << 1) | _lsr(s, 1)
        zr = jnp.zeros((G, 1, W), jnp.int32)
        up = jnp.concatenate([zr, s[:, 0:Hp - 1]], axis=1)
        dn = jnp.concatenate([s[:, 1:Hp], zr], axis=1)
        v = v | (_lsr(up, 31) & 1) | ((dn & 1) << 31)
        zc = jnp.zeros((G, Hp, 1), jnp.int32)
        hp = jnp.concatenate([zc, v, zc], axis=2)
        h = v | hp[:, :, 0:W] | hp[:, :, 2:W + 2]
        return s | (w & h)

    def body(carry):
        _, it = carry
        s = s_ref[...]
        w = w_ref[...]
        new = dilate_once(dilate_once(s, w), w)
        s_ref[...] = new
        changed = jnp.max(new ^ s) > 0
        return changed, it + 2

    def cond(carry):
        changed, it = carry
        return changed & (it < _MAX_HYST_ITERS)

    lax.while_loop(cond, body, (jnp.bool_(True), jnp.int32(0)))


def _lsr(x, k):
    return lax.shift_right_logical(x, jnp.int32(k))


_G = 2


def _canny_body(x_ref, bands_ref, o_ref, s_ref, w_ref):
    H, W = o_ref.shape[2], o_ref.shape[3]
    for g in range(_G):
        strong, weak = _nms_stage(x_ref[g], bands_ref, H, W)
        s_ref[g] = _pack_rows(strong, H, W)
        w_ref[g] = _pack_rows(weak, H, W)
    _hysteresis_loop(s_ref, w_ref, _G, H // 32, W)
    for g in range(_G):
        o_ref[g, 0] = _unpack_rows(s_ref[g], H, W)


def _blur_bands(W):
    g2 = np.array(_gauss2d_bf16(), np.float32)
    bands = np.zeros((_KSIZE, _KPAD, W), np.float32)
    for i in range(_KSIZE):
        for j in range(_KSIZE):
            for c in range(W):
                bands[i, c + j, c] = g2[i, j]
    return bands


@jax.jit
def kernel(x):
    B, C, H, W = x.shape
    bands = jnp.asarray(_blur_bands(W), dtype=jnp.bfloat16)
    out = pl.pallas_call(
        _canny_body,
        grid=(B // _G,),
        in_specs=[pl.BlockSpec((_G, C, H, W), lambda i: (i, 0, 0, 0)),
                  pl.BlockSpec((_KSIZE, _KPAD, W), lambda i: (0, 0, 0))],
        out_specs=pl.BlockSpec((_G, 1, H, W), lambda i: (i, 0, 0, 0)),
        out_shape=jax.ShapeDtypeStruct((B, 1, H, W), jnp.float32),
        scratch_shapes=[
            pltpu.VMEM((_G, H // 32, W), jnp.int32),
            pltpu.VMEM((_G, H // 32, W), jnp.int32),
        ],
        compiler_params=pltpu.CompilerParams(
            dimension_semantics=("parallel",),
            vmem_limit_bytes=64 * 1024 * 1024,
        ),
    )(x, bands)
    return out
